# baseline (speedup 1.0000x reference)
import functools

import jax
import jax.numpy as jnp
from jax import lax
from jax.experimental import pallas as pl
from jax.experimental.pallas import tpu as pltpu
from jax.experimental.pallas import tpu_sc as plsc

_HIDDEN = 128
_VOCAB = 100000

_NC, _NS, _L = 2, 16, 16
_NW = _NC * _NS

_DG_BASE = 4000
_MD_BASE = 84000
_WH_BASE = 116000
_COMB32 = _VOCAB * 4

_WH_G1_N = 12
_WH_G1_PER = 8880
_WH_G1_SIZES = (1000,) * 8 + (880,)
_WH_G2_PER = 8872
_WH_G2_BASE = _WH_BASE + _WH_G1_N * _WH_G1_PER
_WH_G2_SIZES = (1000,) * 8 + (872,)

_DIAG_WORKERS = 25
_DIAG_PER_W = 3200
_DIAG_CHUNK = 800
_MED_PER_W = 32000 // _NW

_RND = jnp.int32(0x8000)
_M_HI = jnp.int32(-65536)

_mesh = plsc.VectorSubcoreMesh(core_axis_name="c", subcore_axis_name="s")
_params = pltpu.CompilerParams(use_tc_tiling_on_sc=False, disable_bounds_checks=True)


def _f32_rows_to_pk(f32_buf, pk_buf, n):
    def body(g8, _):
        for u in range(8):
            g = g8 * 8 + u
            lo = lax.bitcast_convert_type(f32_buf[g, pl.ds(0, _L)], jnp.int32)
            hi = lax.bitcast_convert_type(f32_buf[g, pl.ds(_L, _L)],
                                          jnp.int32)
            pk_buf[g, :] = (lax.shift_right_logical(lo + _RND, 16)
                            | ((hi + _RND) & _M_HI))
        return 0

    lax.fori_loop(0, n // 8, body, 0)


@functools.partial(
    pl.kernel,
    out_type=jax.ShapeDtypeStruct((_COMB32, _L), jnp.int32),
    mesh=_mesh,
    compiler_params=_params,
    scratch_types=[
        pltpu.VMEM((_DIAG_PER_W,), jnp.int32),
        [pltpu.VMEM((1000, 32), jnp.float32) for _ in range(2)],
        [pltpu.VMEM((1000, _L), jnp.int32) for _ in range(2)],
        pltpu.SemaphoreType.DMA,
        pltpu.SemaphoreType.DMA,
        pltpu.SemaphoreType.DMA,
    ],
)
def _build_combined(word32, diag_idx, med_idx, diag_tok, med_tok,
                    comb, idx_v, f32_vs, pk_vs, sem_r, sem_g, sem_w):
    wid = lax.axis_index("s") * _NC + lax.axis_index("c")

    def word_ring(base, sizes):
        offs = [0]
        for s in sizes:
            offs.append(offs[-1] + s)

        def rd(k, b):
            pltpu.async_copy(word32.at[pl.ds(base + offs[k], sizes[k])],
                             f32_vs[b].at[pl.ds(0, sizes[k])], sem_r)

        rd(0, 0)
        if len(sizes) > 1:
            rd(1, 1)
        for k in range(len(sizes)):
            b = k % 2
            pltpu.make_async_copy(
                word32.at[pl.ds(base + offs[k], sizes[k])],
                f32_vs[b].at[pl.ds(0, sizes[k])], sem_r).wait()
            _f32_rows_to_pk(f32_vs[b], pk_vs[b], sizes[k])
            pltpu.async_copy(pk_vs[b].at[pl.ds(0, sizes[k])],
                             comb.at[pl.ds(base + offs[k], sizes[k])], sem_w)
            pltpu.make_async_copy(
                pk_vs[b].at[pl.ds(0, sizes[k])],
                comb.at[pl.ds(base + offs[k], sizes[k])], sem_w).wait()
            if k + 2 < len(sizes):
                rd(k + 2, b)

    @pl.when(wid < _WH_G1_N)
    def _():
        word_ring(_WH_BASE + wid * _WH_G1_PER, _WH_G1_SIZES)

    @pl.when(wid >= _WH_G1_N)
    def _():
        word_ring(_WH_G2_BASE + (wid - _WH_G1_N) * _WH_G2_PER, _WH_G2_SIZES)

    @pl.when((wid >= 28) & (wid < 32))
    def _():
        word_ring((wid - 28) * 1000, (1000,))

    @pl.when(wid < _DIAG_WORKERS)
    def _():
        base = wid * _DIAG_PER_W
        pltpu.sync_copy(diag_idx.at[pl.ds(base, _DIAG_PER_W)], idx_v)

        def fire(k, b):
            pltpu.async_copy(
                diag_tok.at[idx_v.at[pl.ds(k * _DIAG_CHUNK, _DIAG_CHUNK)]],
                f32_vs[b].at[pl.ds(0, _DIAG_CHUNK)], sem_g)

        fire(0, 0)
        fire(1, 1)
        for k in range(4):
            b = k % 2
            pltpu.make_async_copy(
                diag_tok.at[idx_v.at[pl.ds(k * _DIAG_CHUNK, _DIAG_CHUNK)]],
                f32_vs[b].at[pl.ds(0, _DIAG_CHUNK)], sem_g).wait()
            _f32_rows_to_pk(f32_vs[b], pk_vs[b], _DIAG_CHUNK)
            dst = comb.at[pl.ds(_DG_BASE + base + k * _DIAG_CHUNK,
                                _DIAG_CHUNK)]
            pltpu.async_copy(pk_vs[b].at[pl.ds(0, _DIAG_CHUNK)], dst, sem_w)
            pltpu.make_async_copy(
                pk_vs[b].at[pl.ds(0, _DIAG_CHUNK)], dst, sem_w).wait()
            if k + 2 < 4:
                fire(k + 2, b)

    med_base = wid * _MED_PER_W
    pltpu.sync_copy(med_idx.at[pl.ds(med_base, _MED_PER_W)],
                    idx_v.at[pl.ds(0, _MED_PER_W)])
    pltpu.async_copy(med_tok.at[idx_v.at[pl.ds(0, _MED_PER_W)]],
                     f32_vs[0].at[pl.ds(0, _MED_PER_W)], sem_g).wait()
    _f32_rows_to_pk(f32_vs[0], pk_vs[0], _MED_PER_W)
    pltpu.sync_copy(pk_vs[0].at[pl.ds(0, _MED_PER_W)],
                    comb.at[pl.ds(_MD_BASE + med_base, _MED_PER_W)])


def _make_lookup(n_tokens):
    per_w = n_tokens // _NW
    chunk = 256
    sub = 128
    nsub = chunk // sub
    nbuf = 2
    n_chunks = per_w // chunk
    assert n_chunks % nbuf == 0
    idx_load = 6400
    n_idx_loads = per_w // idx_load

    @functools.partial(
        pl.kernel,
        out_type=jax.ShapeDtypeStruct((n_tokens, _HIDDEN), jnp.float32),
        mesh=_mesh,
        compiler_params=_params,
        scratch_types=[
            pltpu.VMEM((per_w,), jnp.int32),
            [pltpu.VMEM((chunk, _HIDDEN // 2), jnp.int32) for _ in range(nbuf)],
            [pltpu.VMEM((chunk, _HIDDEN), jnp.float32) for _ in range(nbuf)],
            pltpu.SemaphoreType.DMA,
            pltpu.SemaphoreType.DMA,
            pltpu.SemaphoreType.DMA,
        ],
    )
    def _lookup(ids, comb, out, idx_all, rows_pk, rows_f, sem_i, sem_g,
                sem_w):
        wid = lax.axis_index("s") * _NC + lax.axis_index("c")
        w_base = wid * per_w

        loads = [
            pltpu.async_copy(
                ids.at[pl.ds(w_base + t * idx_load, idx_load)],
                idx_all.at[pl.ds(t * idx_load, idx_load)], sem_i)
            for t in range(n_idx_loads)
        ]
        for c in loads:
            c.wait()

        def fire_gather(i, b):
            for k in range(nsub):
                pltpu.async_copy(
                    comb.at[idx_all.at[pl.ds(i * chunk + k * sub, sub)]],
                    rows_pk[b].at[pl.ds(k * sub, sub)], sem_g)

        def drain_gather(i, b):
            for k in range(nsub):
                pltpu.make_async_copy(
                    comb.at[idx_all.at[pl.ds(i * chunk + k * sub, sub)]],
                    rows_pk[b].at[pl.ds(k * sub, sub)], sem_g).wait()

        def expand(b):
            def body(r8, _):
                for u in range(8):
                    r = r8 * 8 + u
                    for k in range(_HIDDEN // 32):
                        v = rows_pk[b][r, pl.ds(k * _L, _L)]
                        rows_f[b][r, pl.ds(k * 32, _L)] = (
                            lax.bitcast_convert_type(
                                lax.shift_left(v, 16), jnp.float32))
                        rows_f[b][r, pl.ds(k * 32 + _L, _L)] = (
                            lax.bitcast_convert_type(v & _M_HI, jnp.float32))
                return 0

            lax.fori_loop(0, chunk // 8, body, 0)

        def fire_write(i, b):
            pltpu.async_copy(rows_f[b],
                             out.at[pl.ds(w_base + i * chunk, chunk)], sem_w)

        def drain_write(i, b):
            pltpu.make_async_copy(
                rows_f[b],
                out.at[pl.ds(w_base + i * chunk, chunk)], sem_w).wait()

        for b in range(nbuf):
            fire_gather(b, b)

        def body(jj, _):
            for b in range(nbuf):
                i = jj * nbuf + b
                drain_gather(i, b)
                expand(b)
                fire_write(i, b)
                drain_write(i, b)
                fire_gather(i + nbuf, b)
            return 0

        lax.fori_loop(0, n_chunks // nbuf - 1, body, 0)

        for b in range(nbuf):
            i = n_chunks - nbuf + b
            drain_gather(i, b)
            expand(b)
            fire_write(i, b)
            drain_write(i, b)

    return _lookup


def kernel(input_ids, token_types, diag_tree_table, med_tree_table,
           word_emb, diag_tok, med_tok):
    del token_types
    b, n = input_ids.shape
    ids = input_ids.reshape(-1)
    comb32 = _build_combined(
        word_emb.reshape(_COMB32, 32),
        diag_tree_table.reshape(-1),
        med_tree_table.reshape(-1),
        diag_tok, med_tok)
    comb = comb32.reshape(_VOCAB, _HIDDEN // 2)
    out = _make_lookup(b * n)(ids, comb)
    return out.reshape(b, n, _HIDDEN)


# speedup vs baseline: 1.8923x; 1.8923x over previous
import functools

import jax
import jax.numpy as jnp
from jax import lax
from jax.experimental import pallas as pl
from jax.experimental.pallas import tpu as pltpu
from jax.experimental.pallas import tpu_sc as plsc

_HIDDEN = 128
_VOCAB = 100000

_NC, _NS, _L = 2, 16, 16
_NW = _NC * _NS

_DG_BASE = 4000
_MD_BASE = 84000
_WH_BASE = 116000
_COMB32 = _VOCAB * 4

_WH_G1_N = 12
_WH_G1_PER = 8880
_WH_G1_SIZES = (1800, 1800, 1800, 1800, 1680)
_WH_G2_PER = 8872
_WH_G2_BASE = _WH_BASE + _WH_G1_N * _WH_G1_PER
_WH_G2_SIZES = (1800, 1800, 1800, 1800, 1672)

_DIAG_WORKERS = 25
_DIAG_PER_W = 3200
_DIAG_CHUNK = 1600
_MED_PER_W = 32000 // _NW

_mesh = plsc.VectorSubcoreMesh(core_axis_name="c", subcore_axis_name="s")
_params = pltpu.CompilerParams(use_tc_tiling_on_sc=False)


@functools.partial(
    pl.kernel,
    out_type=jax.ShapeDtypeStruct((_COMB32, 32), jnp.float32),
    mesh=_mesh,
    compiler_params=_params,
    scratch_types=[
        [pltpu.VMEM((_DIAG_CHUNK,), jnp.int32) for _ in range(2)],
        [pltpu.VMEM((1800, 32), jnp.float32) for _ in range(2)],
        pltpu.SemaphoreType.DMA,
        pltpu.SemaphoreType.DMA,
        pltpu.SemaphoreType.DMA,
    ],
)
def _build_combined(word32, diag_idx, med_idx, diag_tok, med_tok,
                    comb, idx_vs, row_vs, sem_r, sem_g, sem_w):
    wid = lax.axis_index("s") * _NC + lax.axis_index("c")

    def word_ring(base, sizes):
        offs = [0]
        for s in sizes:
            offs.append(offs[-1] + s)

        def rd(k, b):
            pltpu.async_copy(
                word32.at[pl.ds(base + offs[k], sizes[k])],
                row_vs[b].at[pl.ds(0, sizes[k])], sem_r)

        rd(0, 0)
        if len(sizes) > 1:
            rd(1, 1)
        for k in range(len(sizes)):
            b = k % 2
            pltpu.make_async_copy(
                word32.at[pl.ds(base + offs[k], sizes[k])],
                row_vs[b].at[pl.ds(0, sizes[k])], sem_r).wait()
            pltpu.async_copy(
                row_vs[b].at[pl.ds(0, sizes[k])],
                comb.at[pl.ds(base + offs[k], sizes[k])], sem_w)
            pltpu.make_async_copy(
                row_vs[b].at[pl.ds(0, sizes[k])],
                comb.at[pl.ds(base + offs[k], sizes[k])], sem_w).wait()
            if k + 2 < len(sizes):
                rd(k + 2, b)

    @pl.when(wid < _WH_G1_N)
    def _():
        word_ring(_WH_BASE + wid * _WH_G1_PER, _WH_G1_SIZES)

    @pl.when(wid >= _WH_G1_N)
    def _():
        word_ring(_WH_G2_BASE + (wid - _WH_G1_N) * _WH_G2_PER, _WH_G2_SIZES)

    @pl.when((wid >= 28) & (wid < 32))
    def _():
        word_ring((wid - 28) * 1000, (1000,))

    @pl.when(wid < _DIAG_WORKERS)
    def _():
        base = wid * _DIAG_PER_W
        for k in range(2):
            pltpu.sync_copy(
                diag_idx.at[pl.ds(base + k * _DIAG_CHUNK, _DIAG_CHUNK)],
                idx_vs[k])
        gs = [pltpu.async_copy(diag_tok.at[idx_vs[k]],
                               row_vs[k].at[pl.ds(0, _DIAG_CHUNK)], sem_g)
              for k in range(2)]
        for k in range(2):
            gs[k].wait()
            pltpu.sync_copy(
                row_vs[k].at[pl.ds(0, _DIAG_CHUNK)],
                comb.at[pl.ds(_DG_BASE + base + k * _DIAG_CHUNK,
                              _DIAG_CHUNK)])

    med_base = wid * _MED_PER_W
    pltpu.sync_copy(med_idx.at[pl.ds(med_base, _MED_PER_W)],
                    idx_vs[0].at[pl.ds(0, _MED_PER_W)])
    pltpu.async_copy(med_tok.at[idx_vs[0].at[pl.ds(0, _MED_PER_W)]],
                     row_vs[0].at[pl.ds(0, _MED_PER_W)], sem_g).wait()
    pltpu.sync_copy(row_vs[0].at[pl.ds(0, _MED_PER_W)],
                    comb.at[pl.ds(_MD_BASE + med_base, _MED_PER_W)])


def _make_lookup(n_tokens):
    per_w = n_tokens // _NW
    chunk = 256
    sub = 128
    nsub = chunk // sub
    nbuf = 2
    n_chunks = per_w // chunk
    assert n_chunks % nbuf == 0
    idx_load = 6400
    n_idx_loads = per_w // idx_load

    @functools.partial(
        pl.kernel,
        out_type=jax.ShapeDtypeStruct((n_tokens, _HIDDEN), jnp.float32),
        mesh=_mesh,
        compiler_params=_params,
        scratch_types=[
            pltpu.VMEM((per_w,), jnp.int32),
            [pltpu.VMEM((chunk, _HIDDEN), jnp.float32) for _ in range(nbuf)],
            pltpu.SemaphoreType.DMA,
            pltpu.SemaphoreType.DMA,
            pltpu.SemaphoreType.DMA,
        ],
    )
    def _lookup(ids, comb, out, idx_all, rows_vs, sem_i, sem_g, sem_w):
        wid = lax.axis_index("s") * _NC + lax.axis_index("c")
        w_base = wid * per_w

        loads = [
            pltpu.async_copy(
                ids.at[pl.ds(w_base + t * idx_load, idx_load)],
                idx_all.at[pl.ds(t * idx_load, idx_load)], sem_i)
            for t in range(n_idx_loads)
        ]
        for c in loads:
            c.wait()

        def fire_gather(i, b):
            for k in range(nsub):
                pltpu.async_copy(
                    comb.at[idx_all.at[pl.ds(i * chunk + k * sub, sub)]],
                    rows_vs[b].at[pl.ds(k * sub, sub)], sem_g)

        def drain_gather(i, b):
            for k in range(nsub):
                pltpu.make_async_copy(
                    comb.at[idx_all.at[pl.ds(i * chunk + k * sub, sub)]],
                    rows_vs[b].at[pl.ds(k * sub, sub)], sem_g).wait()

        def fire_write(i, b):
            pltpu.async_copy(rows_vs[b],
                             out.at[pl.ds(w_base + i * chunk, chunk)], sem_w)

        def drain_write(i, b):
            pltpu.make_async_copy(
                rows_vs[b],
                out.at[pl.ds(w_base + i * chunk, chunk)], sem_w).wait()

        for b in range(nbuf):
            fire_gather(b, b)

        def body(jj, _):
            for b in range(nbuf):
                i = jj * nbuf + b
                drain_gather(i, b)
                fire_write(i, b)
                drain_write(i, b)
                fire_gather(i + nbuf, b)
            return 0

        lax.fori_loop(0, n_chunks // nbuf - 1, body, 0)

        for b in range(nbuf):
            i = n_chunks - nbuf + b
            drain_gather(i, b)
            fire_write(i, b)
            drain_write(i, b)

    return _lookup


def kernel(input_ids, token_types, diag_tree_table, med_tree_table,
           word_emb, diag_tok, med_tok):
    del token_types
    b, n = input_ids.shape
    ids = input_ids.reshape(-1)
    comb32 = _build_combined(
        word_emb.reshape(_COMB32, 32),
        diag_tree_table.reshape(-1),
        med_tree_table.reshape(-1),
        diag_tok, med_tok)
    comb = comb32.reshape(_VOCAB, _HIDDEN)
    out = _make_lookup(b * n)(ids, comb)
    return out.reshape(b, n, _HIDDEN)
